# baseline (speedup 1.0000x reference)
import functools

import numpy as np
import jax
import jax.numpy as jnp
from jax import lax
from jax.experimental import pallas as pl
from jax.experimental.pallas import tpu as pltpu
from jax.experimental.pallas import tpu_sc as plsc

_NODE = 512
_B = 64
_E = 8192
_HEADS = 2
_SLOPE = 0.2
_SELU_SCALE = 1.0507009873554805
_SELU_ALPHA = 1.6732632423543772

_ROWS_PER_TILE = 16
_LANES = 16


def _sc_count_body(src_hbm, dst_hbm, out_hbm, src_v, dst_v, cnt_v):
    lanes_c = np.int32(_LANES)
    node_c = np.int32(_NODE)
    wid = lax.axis_index("s") * np.int32(2) + lax.axis_index("c")
    lo = wid * np.int32(_ROWS_PER_TILE)
    pltpu.sync_copy(src_hbm, src_v)
    pltpu.sync_copy(dst_hbm, dst_v)

    zeros = jnp.zeros((_LANES,), jnp.float32)
    ones = jnp.ones((_LANES,), jnp.float32)
    lane = lax.iota(jnp.int32, _LANES)

    def zero_body(c):
        cnt_v[pl.ds(c * lanes_c, _LANES)] = zeros
        return c + np.int32(1)

    lax.while_loop(lambda c: c < np.int32((_ROWS_PER_TILE * _NODE) // _LANES),
                   zero_body, np.int32(0))

    def edge_body(c):
        s = src_v[pl.ds(c * lanes_c, _LANES)]
        d = dst_v[pl.ds(c * lanes_c, _LANES)]
        rel = d - lo
        inr = (rel >= np.int32(0)) & (rel < np.int32(_ROWS_PER_TILE))
        idx = jnp.clip(rel, np.int32(0), np.int32(_ROWS_PER_TILE - 1)) * node_c + s
        n_in = jnp.sum(inr.astype(jnp.int32),
                       dtype=jnp.int32)

        @pl.when(n_in > np.int32(0))
        def _():
            for j in range(_LANES):
                plsc.addupdate_scatter(cnt_v, [idx], ones, mask=inr & (lane == j))
        return c + np.int32(1)

    lax.while_loop(lambda c: c < np.int32(_E // _LANES), edge_body, np.int32(0))

    diag_idx = lane * node_c + (lo + lane)
    plsc.store_scatter(cnt_v, [diag_idx], ones)

    pltpu.sync_copy(cnt_v, out_hbm.at[pl.ds(wid * np.int32(_ROWS_PER_TILE * _NODE),
                                            _ROWS_PER_TILE * _NODE)])


@functools.cache
def _sc_count_call():
    return pl.kernel(
        _sc_count_body,
        out_type=jax.ShapeDtypeStruct((_NODE * _NODE,), jnp.float32),
        mesh=plsc.VectorSubcoreMesh(core_axis_name="c", subcore_axis_name="s"),
        compiler_params=pltpu.CompilerParams(needs_layout_passes=False),
        scratch_types=[
            pltpu.VMEM((_E,), jnp.int32),
            pltpu.VMEM((_E,), jnp.int32),
            pltpu.VMEM((_ROWS_PER_TILE * _NODE,), jnp.float32),
        ],
    )


def _blockdiag2(a_ref, dout):
    zrow = jnp.zeros((1, dout), jnp.float32)
    return jnp.concatenate(
        [jnp.concatenate([a_ref[pl.ds(0, 1), :], zrow], axis=1),
         jnp.concatenate([zrow, a_ref[pl.ds(1, 1), :]], axis=1)], axis=0)


def _gat_layer(xp, asT, adT, b_ref, C, dout):
    dcols = lax.dot_general(xp, adT, (((1,), (1,)), ((), ())),
                            preferred_element_type=jnp.float32)
    srows = lax.dot_general(asT, xp, (((1,), (1,)), ((), ())),
                            preferred_element_type=jnp.float32)
    ones_col = jnp.ones((_NODE, 1), jnp.bfloat16)
    aggs = []
    denoms = []
    for hh in range(_HEADS):
        xph = xp[:, hh * dout:(hh + 1) * dout]
        s_row = srows[hh:hh + 1, :]
        d_col = dcols[:, hh:hh + 1]
        smax = jnp.max(s_row)
        z = d_col + smax
        u = jnp.exp(s_row - smax).astype(jnp.bfloat16)
        u2 = jnp.exp(_SLOPE * (s_row - smax)).astype(jnp.bfloat16)
        r = jnp.exp(jnp.minimum(-0.8 * z, 80.0)).astype(jnp.bfloat16)
        ex = C * jnp.maximum(u, r * u2)
        xpa = jnp.concatenate([xph.astype(jnp.bfloat16), ones_col], axis=1)
        agg_aug = lax.dot_general(ex, xpa, (((1,), (0,)), ((), ())),
                                  preferred_element_type=jnp.float32)
        aggs.append(agg_aug[:, :dout])
        denoms.append(agg_aug[:, dout:dout + 1])
    rec = pl.reciprocal(jnp.maximum(jnp.concatenate(denoms, axis=1), 1e-30),
                        approx=True)
    h = (0.5 * (aggs[0] * rec[:, 0:1] + aggs[1] * rec[:, 1:2])
         + b_ref[...][None, :])
    return _SELU_SCALE * jnp.where(h > 0, h, _SELU_ALPHA * (jnp.exp(h) - 1.0))


_BPB = 8


def _one_item(xb, nx_part, C, params, wo_ref, bo_ref):
    (w1_ref, w2_ref, W0a, asT0, adT0, b0_ref, asT1, adT1, b1_ref,
     asT2, adT2, b2_ref) = params
    xp0 = (nx_part
           + lax.dot_general(xb, W0a, (((1,), (1,)), ((), ())),
                             preferred_element_type=jnp.float32))
    h = _gat_layer(xp0, asT0, adT0, b0_ref, C, 128)
    xp1 = lax.dot_general(h, w1_ref[...], (((1,), (1,)), ((), ())),
                          preferred_element_type=jnp.float32)
    h = _gat_layer(xp1, asT1, adT1, b1_ref, C, 64)
    xp2 = lax.dot_general(h, w2_ref[...], (((1,), (1,)), ((), ())),
                          preferred_element_type=jnp.float32)
    h = _gat_layer(xp2, asT2, adT2, b2_ref, C, 32)
    return jnp.sum(h * wo_ref[...], axis=1, keepdims=True) + bo_ref[0]


def _tc_body(x_ref, nx_ref, c_ref,
             w0_ref, as0_ref, ad0_ref, b0_ref,
             w1_ref, as1_ref, ad1_ref, b1_ref,
             w2_ref, as2_ref, ad2_ref, b2_ref,
             wo_ref, bo_ref, out_ref):
    g = pl.program_id(0)
    W0 = w0_ref[...]
    Cb = c_ref[...].astype(jnp.bfloat16)
    nx_part = lax.dot_general(nx_ref[...], W0[:, 64:], (((1,), (1,)), ((), ())),
                              preferred_element_type=jnp.float32)
    params = (w1_ref, w2_ref, W0[:, :64],
              _blockdiag2(as0_ref, 128), _blockdiag2(ad0_ref, 128), b0_ref,
              _blockdiag2(as1_ref, 64), _blockdiag2(ad1_ref, 64), b1_ref,
              _blockdiag2(as2_ref, 32), _blockdiag2(ad2_ref, 32), b2_ref)
    for bb in range(_BPB):
        b = g * np.int32(_BPB) + np.int32(bb)
        xb = x_ref[pl.ds(b, 1), :]
        out = _one_item(xb, nx_part, Cb, params, wo_ref, bo_ref)
        out_ref[pl.ds(bb, 1), :, :] = out[None, :, :]


def _full(shape):
    return pl.BlockSpec(shape, lambda b: (np.int32(0),) * len(shape))


_tc_call = pl.pallas_call(
    _tc_body,
    grid=(_B // _BPB,),
    in_specs=[
        _full((_B, 64)), _full((_NODE, 64)), _full((_NODE, _NODE)),
        _full((256, 128)), _full((2, 128)), _full((2, 128)), _full((128,)),
        _full((128, 128)), _full((2, 64)), _full((2, 64)), _full((64,)),
        _full((64, 64)), _full((2, 32)), _full((2, 32)), _full((32,)),
        _full((1, 32)), _full((1,)),
    ],
    out_specs=pl.BlockSpec((_BPB, _NODE, 1),
                           lambda b: (b, np.int32(0), np.int32(0))),
    out_shape=jax.ShapeDtypeStruct((_B, _NODE, 1), jnp.float32),
)


def kernel(x, node_x, edge_index, W0, as0, ad0, b0, W1, as1, ad1, b1,
           W2, as2, ad2, b2, Wout, bout):
    ei = edge_index.astype(jnp.int32)
    counts = _sc_count_call()(ei[0], ei[1])
    C = counts.reshape(_NODE, _NODE)
    out = _tc_call(x.astype(jnp.float32), node_x.astype(jnp.float32), C,
                   W0, as0, ad0, b0, W1, as1, ad1, b1,
                   W2, as2, ad2, b2, Wout, bout)
    return out.reshape(_B, _NODE)


# speedup vs baseline: 1.0308x; 1.0308x over previous
import functools

import numpy as np
import jax
import jax.numpy as jnp
from jax import lax
from jax.experimental import pallas as pl
from jax.experimental.pallas import tpu as pltpu
from jax.experimental.pallas import tpu_sc as plsc

_NODE = 512
_B = 64
_E = 8192
_HEADS = 2
_SLOPE = 0.2
_SELU_SCALE = 1.0507009873554805
_SELU_ALPHA = 1.6732632423543772

_ROWS_PER_TILE = 16
_LANES = 16


def _sc_count_body(src_hbm, dst_hbm, out_hbm, src_v, dst_v, cnt_v):
    lanes_c = np.int32(_LANES)
    node_c = np.int32(_NODE)
    wid = lax.axis_index("s") * np.int32(2) + lax.axis_index("c")
    lo = wid * np.int32(_ROWS_PER_TILE)
    pltpu.sync_copy(src_hbm, src_v)
    pltpu.sync_copy(dst_hbm, dst_v)

    zeros = jnp.zeros((_LANES,), jnp.float32)
    ones = jnp.ones((_LANES,), jnp.float32)
    lane = lax.iota(jnp.int32, _LANES)

    def zero_body(c):
        cnt_v[pl.ds(c * lanes_c, _LANES)] = zeros
        return c + np.int32(1)

    lax.while_loop(lambda c: c < np.int32((_ROWS_PER_TILE * _NODE) // _LANES),
                   zero_body, np.int32(0))

    def edge_body(c):
        s = src_v[pl.ds(c * lanes_c, _LANES)]
        d = dst_v[pl.ds(c * lanes_c, _LANES)]
        rel = d - lo
        inr = (rel >= np.int32(0)) & (rel < np.int32(_ROWS_PER_TILE))
        idx = jnp.clip(rel, np.int32(0), np.int32(_ROWS_PER_TILE - 1)) * node_c + s
        for j in range(_LANES):
            plsc.addupdate_scatter(cnt_v, [idx], ones, mask=inr & (lane == j))
        return c + np.int32(1)

    lax.while_loop(lambda c: c < np.int32(_E // _LANES), edge_body, np.int32(0))

    diag_idx = lane * node_c + (lo + lane)
    plsc.store_scatter(cnt_v, [diag_idx], ones)

    pltpu.sync_copy(cnt_v, out_hbm.at[pl.ds(wid * np.int32(_ROWS_PER_TILE * _NODE),
                                            _ROWS_PER_TILE * _NODE)])


@functools.cache
def _sc_count_call():
    return pl.kernel(
        _sc_count_body,
        out_type=jax.ShapeDtypeStruct((_NODE * _NODE,), jnp.float32),
        mesh=plsc.VectorSubcoreMesh(core_axis_name="c", subcore_axis_name="s"),
        compiler_params=pltpu.CompilerParams(needs_layout_passes=False),
        scratch_types=[
            pltpu.VMEM((_E,), jnp.int32),
            pltpu.VMEM((_E,), jnp.int32),
            pltpu.VMEM((_ROWS_PER_TILE * _NODE,), jnp.float32),
        ],
    )


def _blockdiag2(a_ref, dout):
    zrow = jnp.zeros((1, dout), jnp.float32)
    return jnp.concatenate(
        [jnp.concatenate([a_ref[pl.ds(0, 1), :], zrow], axis=1),
         jnp.concatenate([zrow, a_ref[pl.ds(1, 1), :]], axis=1)], axis=0)


def _gat_layer(xp, asT, adT, b_ref, C, dout):
    dcols = lax.dot_general(xp, adT, (((1,), (1,)), ((), ())),
                            preferred_element_type=jnp.float32)
    srows = lax.dot_general(asT, xp, (((1,), (1,)), ((), ())),
                            preferred_element_type=jnp.float32)
    ones_col = jnp.ones((_NODE, 1), jnp.bfloat16)
    aggs = []
    for hh in range(_HEADS):
        xph = xp[:, hh * dout:(hh + 1) * dout]
        s_row = srows[hh:hh + 1, :]
        d_col = dcols[:, hh:hh + 1]
        smax = jnp.max(s_row)
        z = d_col + smax
        u = jnp.exp(s_row - smax).astype(jnp.bfloat16)
        u2 = jnp.exp(_SLOPE * (s_row - smax)).astype(jnp.bfloat16)
        v = jnp.exp(0.8 * jnp.minimum(z, 0.0)).astype(jnp.bfloat16)
        w2 = jnp.exp(-0.8 * jnp.maximum(z, 0.0)).astype(jnp.bfloat16)
        ex = C * jnp.maximum(v * u, w2 * u2)
        xpa = jnp.concatenate([xph.astype(jnp.bfloat16), ones_col], axis=1)
        agg_aug = lax.dot_general(ex, xpa, (((1,), (0,)), ((), ())),
                                  preferred_element_type=jnp.float32)
        agg = agg_aug[:, :dout]
        denom = agg_aug[:, dout:dout + 1]
        rec = pl.reciprocal(jnp.maximum(denom, 1e-30), approx=True)
        aggs.append(agg * rec)
    h = 0.5 * (aggs[0] + aggs[1]) + b_ref[...][None, :]
    return _SELU_SCALE * jnp.where(h > 0, h, _SELU_ALPHA * (jnp.exp(h) - 1.0))


_BPB = 8


def _one_item(xb, nx_part, C, params, wo_ref, bo_ref):
    (w1_ref, w2_ref, W0a, asT0, adT0, b0_ref, asT1, adT1, b1_ref,
     asT2, adT2, b2_ref) = params
    xp0 = (nx_part
           + lax.dot_general(xb, W0a, (((1,), (1,)), ((), ())),
                             preferred_element_type=jnp.float32))
    h = _gat_layer(xp0, asT0, adT0, b0_ref, C, 128)
    xp1 = lax.dot_general(h, w1_ref[...], (((1,), (1,)), ((), ())),
                          preferred_element_type=jnp.float32)
    h = _gat_layer(xp1, asT1, adT1, b1_ref, C, 64)
    xp2 = lax.dot_general(h, w2_ref[...], (((1,), (1,)), ((), ())),
                          preferred_element_type=jnp.float32)
    h = _gat_layer(xp2, asT2, adT2, b2_ref, C, 32)
    return jnp.sum(h * wo_ref[...], axis=1, keepdims=True) + bo_ref[0]


def _tc_body(x_ref, nx_ref, c_ref,
             w0_ref, as0_ref, ad0_ref, b0_ref,
             w1_ref, as1_ref, ad1_ref, b1_ref,
             w2_ref, as2_ref, ad2_ref, b2_ref,
             wo_ref, bo_ref, out_ref):
    g = pl.program_id(0)
    W0 = w0_ref[...]
    Cb = c_ref[...].astype(jnp.bfloat16)
    nx_part = lax.dot_general(nx_ref[...], W0[:, 64:], (((1,), (1,)), ((), ())),
                              preferred_element_type=jnp.float32)
    params = (w1_ref, w2_ref, W0[:, :64],
              _blockdiag2(as0_ref, 128), _blockdiag2(ad0_ref, 128), b0_ref,
              _blockdiag2(as1_ref, 64), _blockdiag2(ad1_ref, 64), b1_ref,
              _blockdiag2(as2_ref, 32), _blockdiag2(ad2_ref, 32), b2_ref)
    for bb in range(_BPB):
        b = g * np.int32(_BPB) + np.int32(bb)
        xb = x_ref[pl.ds(b, 1), :]
        out = _one_item(xb, nx_part, Cb, params, wo_ref, bo_ref)
        out_ref[pl.ds(bb, 1), :, :] = out[None, :, :]


def _full(shape):
    return pl.BlockSpec(shape, lambda b: (np.int32(0),) * len(shape))


_tc_call = pl.pallas_call(
    _tc_body,
    grid=(_B // _BPB,),
    in_specs=[
        _full((_B, 64)), _full((_NODE, 64)), _full((_NODE, _NODE)),
        _full((256, 128)), _full((2, 128)), _full((2, 128)), _full((128,)),
        _full((128, 128)), _full((2, 64)), _full((2, 64)), _full((64,)),
        _full((64, 64)), _full((2, 32)), _full((2, 32)), _full((32,)),
        _full((1, 32)), _full((1,)),
    ],
    out_specs=pl.BlockSpec((_BPB, _NODE, 1),
                           lambda b: (b, np.int32(0), np.int32(0))),
    out_shape=jax.ShapeDtypeStruct((_B, _NODE, 1), jnp.float32),
)


def kernel(x, node_x, edge_index, W0, as0, ad0, b0, W1, as1, ad1, b1,
           W2, as2, ad2, b2, Wout, bout):
    ei = edge_index.astype(jnp.int32)
    counts = _sc_count_call()(ei[0], ei[1])
    C = counts.reshape(_NODE, _NODE)
    out = _tc_call(x.astype(jnp.float32), node_x.astype(jnp.float32), C,
                   W0, as0, ad0, b0, W1, as1, ad1, b1,
                   W2, as2, ad2, b2, Wout, bout)
    return out.reshape(_B, _NODE)
